# baseline (speedup 1.0000x reference)
import jax
import jax.numpy as jnp
import numpy as np
from jax import lax
from jax.experimental import pallas as pl
from jax.experimental.pallas import tpu as pltpu
from jax.experimental.pallas import tpu_sc as plsc

SIZE = 256
CAMERA_RANGE = 3.0
AGENT_RADIUS = 0.05
OBSTACLE_BASE_HEIGHT = 0.5
HALF_FOV = float(np.radians(90.0)) / 2.0
NOBJ = 159
NPAD = 160
NGROUPS = NPAD // 16
NTILES = 32
ROWS_PER_TILE = SIZE // NTILES
LANES = 16


def _paint_body(obj_hbm, out_hbm, obj_v, rows_v):
    cid = lax.axis_index("c")
    sid = lax.axis_index("s")
    wid = sid * 2 + cid

    pltpu.sync_copy(obj_hbm, obj_v)

    ones = jnp.full((LANES,), 1.0, jnp.float32)
    for j in range(ROWS_PER_TILE):
        for sg in range(SIZE // LANES):
            rows_v[j, pl.ds(sg * LANES, LANES)] = ones

    iota = lax.convert_element_type(lax.iota(jnp.int32, LANES), jnp.float32)

    def paint_row(j, px16, pr16, pr216, dv16, clo16, vlo16):
        r = wid + NTILES * j
        dy = lax.convert_element_type(r, jnp.float32) - 128.0
        dy2 = dy * dy
        w216 = pr216 - dy2
        chi16 = jnp.where(w216 >= 0.0,
                          jnp.minimum(px16 + pr16, 255.0), -16.0)
        vhi16 = lax.convert_element_type(chi16 * 0.0625, jnp.int32)
        for k in range(LANES):
            vlo_k = vlo16[k]
            vhi_k = vhi16[k]

            @pl.when(vhi_k >= vlo_k)
            def _lane():
                px_o = px16[k]
                dv_o = dv16[k]
                w2_o = pr216[k] - dy2

                def seg_body(vb, c2):
                    base = vb * LANES
                    cols = lax.convert_element_type(base, jnp.float32) + iota
                    dx = cols - px_o
                    m = dx * dx <= w2_o
                    seg = rows_v[j, pl.ds(base, LANES)]
                    rows_v[j, pl.ds(base, LANES)] = jnp.where(
                        m, jnp.minimum(seg, dv_o), seg)
                    return c2

                lax.fori_loop(vlo_k, vhi_k + 1, seg_body, 0)

    def group_body(g, carry):
        base_o = g * LANES
        px16 = obj_v[0, pl.ds(base_o, LANES)]
        pr16 = obj_v[1, pl.ds(base_o, LANES)]
        pr216 = obj_v[2, pl.ds(base_o, LANES)]
        dv16 = obj_v[3, pl.ds(base_o, LANES)]
        clo16 = jnp.maximum(px16 - pr16, 0.0)
        vlo16 = lax.convert_element_type(clo16 * 0.0625, jnp.int32)
        for j in (2, 3, 4, 5):
            paint_row(j, px16, pr16, pr216, dv16, clo16, vlo16)

        @pl.when(wid == 0)
        def _row192():
            paint_row(6, px16, pr16, pr216, dv16, clo16, vlo16)

        return carry

    lax.fori_loop(0, NGROUPS, group_body, 0)

    for j in range(ROWS_PER_TILE):
        r = wid + NTILES * j
        pltpu.sync_copy(rows_v.at[j], out_hbm.at[0, r])


_paint = pl.kernel(
    _paint_body,
    out_type=jax.ShapeDtypeStruct((1, SIZE, SIZE), jnp.float32),
    mesh=plsc.VectorSubcoreMesh(core_axis_name="c", subcore_axis_name="s"),
    scratch_types=[
        pltpu.VMEM((4, NPAD), jnp.float32),
        pltpu.VMEM((ROWS_PER_TILE, SIZE), jnp.float32),
    ],
)


def _bf16_rne(x):
    u = jax.lax.bitcast_convert_type(x, jnp.uint32)
    lsb = (u >> 16) & jnp.uint32(1)
    u = (u + jnp.uint32(0x7FFF) + lsb) & jnp.uint32(0xFFFF0000)
    return jax.lax.bitcast_convert_type(u, jnp.float32)


def kernel(agent_pos, goal_pos, other_agents, obstacles):
    vd = goal_pos - agent_pos
    vd = vd / (jnp.linalg.norm(vd) + 1e-08)
    cos_t = vd[1]
    sin_t = vd[0]

    pos = jnp.concatenate([other_agents, obstacles[:, :2]], axis=0)
    radius = jnp.concatenate([
        jnp.full((other_agents.shape[0],), AGENT_RADIUS, jnp.float32),
        obstacles[:, 2],
    ])
    height = jnp.concatenate([
        jnp.full((other_agents.shape[0],), 0.2, jnp.float32),
        jnp.full((obstacles.shape[0],), OBSTACLE_BASE_HEIGHT, jnp.float32),
    ])

    rel0 = pos[:, 0] - agent_pos[0]
    rel1 = pos[:, 1] - agent_pos[1]
    bc = _bf16_rne(cos_t)
    bs = _bf16_rne(sin_t)
    bns = _bf16_rne(-sin_t)
    br0 = _bf16_rne(rel0)
    br1 = _bf16_rne(rel1)
    cam0 = bc * br0 + bs * br1
    cam1 = bns * br0 + bc * br1
    fdot = rel0 * (-sin_t) + rel1 * cos_t
    dist = jnp.sqrt(cam0 * cam0 + cam1 * cam1)
    angle_x = jnp.arctan2(cam0, cam1)
    visible = (fdot >= 0.0) & (dist <= CAMERA_RANGE) & (jnp.abs(angle_x) <= HALF_FOV)
    pixel_x = angle_x / HALF_FOV * 0.5
    px = jnp.floor((pixel_x + 0.5) * SIZE)
    pr = jnp.floor(radius / (dist + 1e-08) * SIZE * 0.5)
    pr = jnp.clip(pr, 1.0, float(SIZE // 4))
    dval = jnp.minimum(dist / CAMERA_RANGE, 1.0)
    dval = dval * (1.0 - height * 0.3)
    dval = jnp.maximum(dval, 0.0)

    obj = jnp.stack([
        jnp.where(visible, px, 0.0),
        jnp.where(visible, pr, 0.0),
        jnp.where(visible, pr * pr, -1.0),
        jnp.where(visible, dval, 0.0),
    ])
    obj = jnp.pad(obj, ((0, 0), (0, NPAD - NOBJ)),
                  constant_values=-1.0)
    return _paint(obj)


# speedup vs baseline: 1.7343x; 1.7343x over previous
import jax
import jax.numpy as jnp
import numpy as np
from jax import lax
from jax.experimental import pallas as pl
from jax.experimental.pallas import tpu as pltpu
from jax.experimental.pallas import tpu_sc as plsc

SIZE = 256
CAMERA_RANGE = 3.0
AGENT_RADIUS = 0.05
OBSTACLE_BASE_HEIGHT = 0.5
HALF_FOV = float(np.radians(90.0)) / 2.0
NOBJ = 159
NPAD = 164
NTILES = 32
ROWS_PER_TILE = SIZE // NTILES
LANES = 16


def _paint_body(obj_hbm, out_hbm, obj_v, rows_v):
    cid = lax.axis_index("c")
    sid = lax.axis_index("s")
    wid = sid * 2 + cid

    pltpu.sync_copy(obj_hbm, obj_v)

    ones = jnp.full((LANES,), 1.0, jnp.float32)
    for j in range(ROWS_PER_TILE):
        for sg in range(SIZE // LANES):
            rows_v[j, pl.ds(sg * LANES, LANES)] = ones

    iota = lax.convert_element_type(lax.iota(jnp.int32, LANES), jnp.float32)

    BAND_J = (2, 3, 4, 5)
    widf = lax.convert_element_type(wid, jnp.float32)
    dy2_j = [(widf + float(NTILES * j - 128)) * (widf + float(NTILES * j - 128))
             for j in BAND_J]
    mindy2 = jnp.minimum(jnp.minimum(dy2_j[0], dy2_j[1]),
                         jnp.minimum(dy2_j[2], dy2_j[3]))

    def paint_obj_row(j, dy2, px_o, dv_o, pr2_o, vlo, vhi):
        w2 = pr2_o - dy2

        @pl.when(w2 >= 0.0)
        def _row():
            def seg_body(vb, c2):
                base = vb * LANES
                cols = lax.convert_element_type(base, jnp.float32) + iota
                dx = cols - px_o
                m = dx * dx <= w2
                seg = rows_v[j, pl.ds(base, LANES)]
                rows_v[j, pl.ds(base, LANES)] = jnp.where(
                    m, jnp.minimum(seg, dv_o), seg)
                return c2

            lax.fori_loop(vlo, vhi + 1, seg_body, 0)

    def obj_body(o, carry):
        vec = obj_v[pl.ds(o * 4, LANES)]
        pr2_o = vec[2]

        @pl.when(pr2_o >= mindy2)
        def _obj():
            px_o = vec[0]
            pr_o = vec[1]
            dv_o = vec[3]
            clo = jnp.maximum(px_o - pr_o, 0.0)
            chi = jnp.minimum(px_o + pr_o, 255.0)
            vlo = lax.convert_element_type(clo * 0.0625, jnp.int32)
            vhi = lax.convert_element_type(chi * 0.0625, jnp.int32)
            for idx, j in enumerate(BAND_J):
                paint_obj_row(j, dy2_j[idx], px_o, dv_o, pr2_o, vlo, vhi)

        return carry

    lax.fori_loop(0, NOBJ, obj_body, 0)

    @pl.when(wid == 0)
    def _row192():
        def obj_body6(o, carry):
            vec = obj_v[pl.ds(o * 4, LANES)]
            pr2_o = vec[2]

            @pl.when(pr2_o >= 4096.0)
            def _obj():
                px_o = vec[0]
                pr_o = vec[1]
                dv_o = vec[3]
                clo = jnp.maximum(px_o - pr_o, 0.0)
                chi = jnp.minimum(px_o + pr_o, 255.0)
                vlo = lax.convert_element_type(clo * 0.0625, jnp.int32)
                vhi = lax.convert_element_type(chi * 0.0625, jnp.int32)
                paint_obj_row(6, 4096.0, px_o, dv_o, pr2_o, vlo, vhi)

            return carry

        lax.fori_loop(0, NOBJ, obj_body6, 0)

    for j in range(ROWS_PER_TILE):
        r = wid + NTILES * j
        pltpu.sync_copy(rows_v.at[j], out_hbm.at[0, r])


_paint = pl.kernel(
    _paint_body,
    out_type=jax.ShapeDtypeStruct((1, SIZE, SIZE), jnp.float32),
    mesh=plsc.VectorSubcoreMesh(core_axis_name="c", subcore_axis_name="s"),
    scratch_types=[
        pltpu.VMEM((4 * NPAD,), jnp.float32),
        pltpu.VMEM((ROWS_PER_TILE, SIZE), jnp.float32),
    ],
)


def _bf16_rne(x):
    u = jax.lax.bitcast_convert_type(x, jnp.uint32)
    lsb = (u >> 16) & jnp.uint32(1)
    u = (u + jnp.uint32(0x7FFF) + lsb) & jnp.uint32(0xFFFF0000)
    return jax.lax.bitcast_convert_type(u, jnp.float32)


def kernel(agent_pos, goal_pos, other_agents, obstacles):
    vd = goal_pos - agent_pos
    vd = vd / (jnp.linalg.norm(vd) + 1e-08)
    cos_t = vd[1]
    sin_t = vd[0]

    pos = jnp.concatenate([other_agents, obstacles[:, :2]], axis=0)
    radius = jnp.concatenate([
        jnp.full((other_agents.shape[0],), AGENT_RADIUS, jnp.float32),
        obstacles[:, 2],
    ])
    height = jnp.concatenate([
        jnp.full((other_agents.shape[0],), 0.2, jnp.float32),
        jnp.full((obstacles.shape[0],), OBSTACLE_BASE_HEIGHT, jnp.float32),
    ])

    rel0 = pos[:, 0] - agent_pos[0]
    rel1 = pos[:, 1] - agent_pos[1]
    bc = _bf16_rne(cos_t)
    bs = _bf16_rne(sin_t)
    bns = _bf16_rne(-sin_t)
    br0 = _bf16_rne(rel0)
    br1 = _bf16_rne(rel1)
    cam0 = bc * br0 + bs * br1
    cam1 = bns * br0 + bc * br1
    fdot = rel0 * (-sin_t) + rel1 * cos_t
    dist = jnp.sqrt(cam0 * cam0 + cam1 * cam1)
    angle_x = jnp.arctan2(cam0, cam1)
    visible = (fdot >= 0.0) & (dist <= CAMERA_RANGE) & (jnp.abs(angle_x) <= HALF_FOV)
    pixel_x = angle_x / HALF_FOV * 0.5
    px = jnp.floor((pixel_x + 0.5) * SIZE)
    pr = jnp.floor(radius / (dist + 1e-08) * SIZE * 0.5)
    pr = jnp.clip(pr, 1.0, float(SIZE // 4))
    dval = jnp.minimum(dist / CAMERA_RANGE, 1.0)
    dval = dval * (1.0 - height * 0.3)
    dval = jnp.maximum(dval, 0.0)

    obj = jnp.stack([
        jnp.where(visible, px, 0.0),
        jnp.where(visible, pr, 0.0),
        jnp.where(visible, pr * pr, -1.0),
        jnp.where(visible, dval, 0.0),
    ], axis=1)
    obj = jnp.pad(obj, ((0, NPAD - NOBJ), (0, 0)),
                  constant_values=-1.0)
    return _paint(obj.reshape(4 * NPAD))


# speedup vs baseline: 1.7354x; 1.0006x over previous
import jax
import jax.numpy as jnp
import numpy as np
from jax import lax
from jax.experimental import pallas as pl
from jax.experimental.pallas import tpu as pltpu
from jax.experimental.pallas import tpu_sc as plsc

SIZE = 256
CAMERA_RANGE = 3.0
AGENT_RADIUS = 0.05
OBSTACLE_BASE_HEIGHT = 0.5
HALF_FOV = float(np.radians(90.0)) / 2.0
NOBJ = 159
NPAD = 164
NTILES = 32
ROWS_PER_TILE = SIZE // NTILES
LANES = 16


def _paint_body(obj_hbm, out_hbm, obj_v, rows_v):
    cid = lax.axis_index("c")
    sid = lax.axis_index("s")
    wid = sid * 2 + cid

    pltpu.sync_copy(obj_hbm, obj_v)

    ones = jnp.full((LANES,), 1.0, jnp.float32)
    for j in range(ROWS_PER_TILE):
        for sg in range(SIZE // LANES):
            rows_v[j, pl.ds(sg * LANES, LANES)] = ones

    iota = lax.convert_element_type(lax.iota(jnp.int32, LANES), jnp.float32)

    BAND_J = (2, 3, 4, 5)
    widf = lax.convert_element_type(wid, jnp.float32)
    dy2_j = [(widf + float(NTILES * j - 128)) * (widf + float(NTILES * j - 128))
             for j in BAND_J]
    mindy2 = jnp.minimum(jnp.minimum(dy2_j[0], dy2_j[1]),
                         jnp.minimum(dy2_j[2], dy2_j[3]))

    def paint_obj_row(j, dy2, px_o, dv_o, pr2_o, vlo, vhi):
        w2 = pr2_o - dy2

        @pl.when(w2 >= 0.0)
        def _row():
            def seg_body(vb, c2):
                base = vb * LANES
                cols = lax.convert_element_type(base, jnp.float32) + iota
                dx = cols - px_o
                m = dx * dx <= w2
                seg = rows_v[j, pl.ds(base, LANES)]
                rows_v[j, pl.ds(base, LANES)] = jnp.where(
                    m, jnp.minimum(seg, dv_o), seg)
                return c2

            lax.fori_loop(vlo, vhi + 1, seg_body, 0)

    def obj_body(o, carry):
        vec = obj_v[pl.ds(o * 4, LANES)]
        pr2_o = vec[2]

        @pl.when(pr2_o >= mindy2)
        def _obj():
            px_o = vec[0]
            pr_o = vec[1]
            dv_o = vec[3]
            clo = jnp.maximum(px_o - pr_o, 0.0)
            chi = jnp.minimum(px_o + pr_o, 255.0)
            vlo = lax.convert_element_type(clo, jnp.int32) >> 4
            vhi = lax.convert_element_type(chi, jnp.int32) >> 4
            for idx, j in enumerate(BAND_J):
                paint_obj_row(j, dy2_j[idx], px_o, dv_o, pr2_o, vlo, vhi)

        return carry

    lax.fori_loop(0, NOBJ, obj_body, 0)

    @pl.when(wid == 0)
    def _row192():
        def obj_body6(o, carry):
            vec = obj_v[pl.ds(o * 4, LANES)]
            pr2_o = vec[2]

            @pl.when(pr2_o >= 4096.0)
            def _obj():
                px_o = vec[0]
                pr_o = vec[1]
                dv_o = vec[3]
                clo = jnp.maximum(px_o - pr_o, 0.0)
                chi = jnp.minimum(px_o + pr_o, 255.0)
                vlo = lax.convert_element_type(clo, jnp.int32) >> 4
                vhi = lax.convert_element_type(chi, jnp.int32) >> 4
                paint_obj_row(6, 4096.0, px_o, dv_o, pr2_o, vlo, vhi)

            return carry

        lax.fori_loop(0, NOBJ, obj_body6, 0)

    for j in range(ROWS_PER_TILE):
        r = wid + NTILES * j
        pltpu.sync_copy(rows_v.at[j], out_hbm.at[0, r])


_paint = pl.kernel(
    _paint_body,
    out_type=jax.ShapeDtypeStruct((1, SIZE, SIZE), jnp.float32),
    mesh=plsc.VectorSubcoreMesh(core_axis_name="c", subcore_axis_name="s"),
    scratch_types=[
        pltpu.VMEM((4 * NPAD,), jnp.float32),
        pltpu.VMEM((ROWS_PER_TILE, SIZE), jnp.float32),
    ],
)


def _bf16_rne(x):
    u = jax.lax.bitcast_convert_type(x, jnp.uint32)
    lsb = (u >> 16) & jnp.uint32(1)
    u = (u + jnp.uint32(0x7FFF) + lsb) & jnp.uint32(0xFFFF0000)
    return jax.lax.bitcast_convert_type(u, jnp.float32)


def kernel(agent_pos, goal_pos, other_agents, obstacles):
    vd = goal_pos - agent_pos
    vd = vd / (jnp.linalg.norm(vd) + 1e-08)
    cos_t = vd[1]
    sin_t = vd[0]

    pos = jnp.concatenate([other_agents, obstacles[:, :2]], axis=0)
    radius = jnp.concatenate([
        jnp.full((other_agents.shape[0],), AGENT_RADIUS, jnp.float32),
        obstacles[:, 2],
    ])
    height = jnp.concatenate([
        jnp.full((other_agents.shape[0],), 0.2, jnp.float32),
        jnp.full((obstacles.shape[0],), OBSTACLE_BASE_HEIGHT, jnp.float32),
    ])

    rel0 = pos[:, 0] - agent_pos[0]
    rel1 = pos[:, 1] - agent_pos[1]
    bc = _bf16_rne(jnp.broadcast_to(cos_t, rel0.shape))
    bs = _bf16_rne(jnp.broadcast_to(sin_t, rel0.shape))
    bns = _bf16_rne(jnp.broadcast_to(-sin_t, rel0.shape))
    br0 = _bf16_rne(rel0)
    br1 = _bf16_rne(rel1)
    cam0 = bc * br0 + bs * br1
    cam1 = bns * br0 + bc * br1
    fdot = rel0 * (-sin_t) + rel1 * cos_t
    dist = jnp.sqrt(cam0 * cam0 + cam1 * cam1)
    angle_x = jnp.arctan2(cam0, cam1)
    visible = (fdot >= 0.0) & (dist <= CAMERA_RANGE) & (jnp.abs(angle_x) <= HALF_FOV)
    pixel_x = angle_x / HALF_FOV * 0.5
    px = jnp.floor((pixel_x + 0.5) * SIZE)
    pr = jnp.floor(radius / (dist + 1e-08) * SIZE * 0.5)
    pr = jnp.clip(pr, 1.0, float(SIZE // 4))
    dval = jnp.minimum(dist / CAMERA_RANGE, 1.0)
    dval = dval * (1.0 - height * 0.3)
    dval = jnp.maximum(dval, 0.0)

    obj = jnp.stack([
        jnp.where(visible, px, 0.0),
        jnp.where(visible, pr, 0.0),
        jnp.where(visible, pr * pr, -1.0),
        jnp.where(visible, dval, 0.0),
    ], axis=1)
    obj = jnp.pad(obj, ((0, NPAD - NOBJ), (0, 0)),
                  constant_values=-1.0)
    return _paint(obj.reshape(4 * NPAD))
